# baseline (speedup 1.0000x reference)
import functools

import jax
import jax.numpy as jnp
from jax import lax
from jax.experimental import pallas as pl
from jax.experimental.pallas import tpu as pltpu
from jax.experimental.pallas import tpu_sc as plsc


def _sc_half(xf, token_table, pos_table, n_sc):
    vocab, dim = token_table.shape
    seq, _ = pos_table.shape

    mesh = plsc.VectorSubcoreMesh(core_axis_name="c", subcore_axis_name="s")
    nw = mesh.num_cores * mesh.num_subcores
    n_per_w = n_sc // nw
    assert n_sc % nw == 0
    assert n_per_w % 16 == 0 and n_per_w % 8 == 0 and dim % 16 == 0
    pos_gran = __import__("math").gcd(n_per_w, seq)
    assert pos_gran % 8 == 0

    @functools.partial(
        pl.kernel,
        out_type=jax.ShapeDtypeStruct((n_sc, dim), jnp.float32),
        mesh=mesh,
        compiler_params=pltpu.CompilerParams(
            use_tc_tiling_on_sc=True, needs_layout_passes=False
        ),
        scratch_types=[
            pltpu.VMEM((n_per_w,), jnp.int32),
            pltpu.VMEM((8, dim, 128), jnp.float32),
            pltpu.VMEM((n_per_w, dim), jnp.float32),
            pltpu.SemaphoreType.DMA,
            pltpu.SemaphoreType.DMA,
            pltpu.SemaphoreType.DMA,
        ],
    )
    def _emb(idx_hbm, tbl_hbm, pos_hbm, out_hbm, idx_v, blk_v, rows_v,
             psem, sa, sb):
        wid = lax.axis_index("s") * mesh.num_cores + lax.axis_index("c")
        base = wid * n_per_w
        pos_base = lax.rem(base, seq)
        pltpu.sync_copy(idx_hbm.at[pl.ds(base, n_per_w)], idx_v)

        @pl.when(pos_base <= seq - n_per_w)
        def _pos_nowrap():
            pltpu.sync_copy(pos_hbm.at[pl.ds(pos_base, n_per_w)], rows_v)

        for first in range(pos_gran, n_per_w, pos_gran):
            @pl.when(pos_base == seq - first)
            def _pos_wrap(first=first):
                pltpu.sync_copy(
                    pos_hbm.at[pl.ds(seq - first, first)],
                    rows_v.at[pl.ds(0, first)],
                )
                pltpu.sync_copy(
                    pos_hbm.at[pl.ds(0, n_per_w - first)],
                    rows_v.at[pl.ds(first, n_per_w - first)],
                )

        sems = [sa, sb]
        dvecs = [lax.iota(jnp.int32, 16) + 16 * c for c in range(dim // 16)]
        nq = n_per_w // 4
        vs = [idx_v[pl.ds(g * 16, 16)] for g in range(n_per_w // 16)]

        def fire(q):
            slab = q % 2
            for j in range(4):
                i = 4 * q + j
                v = vs[i // 16][i % 16]
                vb = pl.multiple_of((v >> 7) * 128, 128)
                pltpu.async_copy(
                    tbl_hbm.at[:, pl.ds(vb, 128)],
                    blk_v.at[4 * slab + j],
                    sems[slab],
                )

        def wait_quad(q):
            for _ in range(4):
                pltpu.make_async_copy(
                    tbl_hbm.at[:, pl.ds(0, 128)], blk_v.at[0], sems[q % 2]
                ).wait()

        def extract(q):
            slab = q % 2
            for j in range(4):
                i = 4 * q + j
                v = vs[i // 16][i % 16]
                p = jnp.full((16,), v & 127, jnp.int32)
                for c in range(dim // 16):
                    tok = plsc.load_gather(blk_v.at[4 * slab + j], [dvecs[c], p])
                    sl = (i, pl.ds(16 * c, 16))
                    rows_v[sl] = rows_v[sl] + tok

        fire(0)
        fire(1)
        for q in range(nq):
            wait_quad(q)
            extract(q)
            if q + 2 < nq:
                fire(q + 2)

        pltpu.sync_copy(rows_v, out_hbm.at[pl.ds(base, n_per_w)])

    return _emb(xf, token_table.T, pos_table)


def _tc_half(idx_tc, tbl_t, pos_table, n_sc):
    dim, vocab = tbl_t.shape
    seq, _ = pos_table.shape
    n_tc = idx_tc.shape[0]
    K = 16
    steps = n_tc // K
    assert n_tc % K == 0 and seq % K == 0 and n_sc % K == 0

    def body(idx_sref, tbl_ref, pos_ref, out_ref, blk, s0, s1):
        i = pl.program_id(0)
        sems = [s0, s1]

        def issue(step, par):
            for j in range(K):
                v = idx_sref[step * K + j]
                vb = pl.multiple_of((v >> 7) * 128, 128)
                pltpu.make_async_copy(
                    tbl_ref.at[:, pl.ds(vb, 128)], blk.at[par, j], sems[par]
                ).start()

        def drain(par):
            for _ in range(K):
                pltpu.make_async_copy(
                    tbl_ref.at[:, pl.ds(0, 128)], blk.at[0, 0], sems[par]
                ).wait()

        def extract(par):
            drain(par)
            cols = []
            for j in range(K):
                p = idx_sref[i * K + j] & 127
                rolled = pltpu.roll(blk[par, j], -p, 1)
                cols.append(rolled[:, 0:1])
            ext = jnp.concatenate(cols, axis=1)
            out_ref[...] = ext.T + pos_ref[...]

        @pl.when(i == 0)
        def _prime():
            issue(0, 0)

        @pl.when(jnp.logical_and(i + 1 < steps, (i + 1) % 2 == 1))
        def _next_odd():
            issue(i + 1, 1)

        @pl.when(jnp.logical_and(i + 1 < steps, (i + 1) % 2 == 0))
        def _next_even():
            issue(i + 1, 0)

        @pl.when(i % 2 == 0)
        def _cur_even():
            extract(0)

        @pl.when(i % 2 == 1)
        def _cur_odd():
            extract(1)

    grid_spec = pltpu.PrefetchScalarGridSpec(
        num_scalar_prefetch=1,
        grid=(steps,),
        in_specs=[
            pl.BlockSpec(memory_space=pl.ANY),
            pl.BlockSpec(
                (K, dim),
                lambda i, idx_ref: ((n_sc // K + i) % (seq // K), 0),
            ),
        ],
        out_specs=pl.BlockSpec((K, dim), lambda i, idx_ref: (i, 0)),
        scratch_shapes=[
            pltpu.VMEM((2, K, dim, 128), jnp.float32),
            pltpu.SemaphoreType.DMA,
            pltpu.SemaphoreType.DMA,
        ],
    )
    return pl.pallas_call(
        body,
        grid_spec=grid_spec,
        out_shape=jax.ShapeDtypeStruct((n_tc, dim), jnp.float32),
    )(idx_tc, tbl_t, pos_table)


def kernel(x, token_table, pos_table):
    batch, seq = x.shape
    vocab, dim = token_table.shape
    n = batch * seq
    n_sc = 11 * n // 16

    xf = x.reshape(n)
    out_sc = _sc_half(xf, token_table, pos_table, n_sc)
    out_tc = _tc_half(xf[n_sc:], token_table.T, pos_table, n_sc)
    out = jnp.concatenate([out_sc, out_tc], axis=0)
    return out.reshape(batch, seq, dim)


# speedup vs baseline: 1.2628x; 1.2628x over previous
import functools
import math

import jax
import jax.numpy as jnp
from jax import lax
from jax.experimental import pallas as pl
from jax.experimental.pallas import tpu as pltpu
from jax.experimental.pallas import tpu_sc as plsc


def _sc_half(xf, token_table, pos_table, n_sc):
    vocab, dim = token_table.shape
    seq, _ = pos_table.shape

    mesh = plsc.VectorSubcoreMesh(core_axis_name="c", subcore_axis_name="s")
    nw = mesh.num_cores * mesh.num_subcores
    n_per_w = n_sc // nw
    assert n_sc % nw == 0
    assert n_per_w % 16 == 0 and n_per_w % 8 == 0 and dim % 16 == 0
    pos_gran = math.gcd(n_per_w, seq)
    assert pos_gran % 8 == 0

    @functools.partial(
        pl.kernel,
        out_type=jax.ShapeDtypeStruct((n_sc, dim), jnp.float32),
        mesh=mesh,
        compiler_params=pltpu.CompilerParams(
            use_tc_tiling_on_sc=True, needs_layout_passes=False
        ),
        scratch_types=[
            pltpu.VMEM((n_per_w,), jnp.int32),
            pltpu.VMEM((8, dim, 128), jnp.float32),
            pltpu.VMEM((n_per_w, dim), jnp.float32),
            pltpu.SemaphoreType.DMA,
            pltpu.SemaphoreType.DMA,
            pltpu.SemaphoreType.DMA,
        ],
    )
    def _emb(idx_hbm, tbl_hbm, pos_hbm, out_hbm, idx_v, blk_v, rows_v,
             psem, sa, sb):
        wid = lax.axis_index("s") * mesh.num_cores + lax.axis_index("c")
        base = wid * n_per_w
        pos_base = lax.rem(base, seq)
        pltpu.sync_copy(idx_hbm.at[pl.ds(base, n_per_w)], idx_v)

        @pl.when(pos_base <= seq - n_per_w)
        def _pos_nowrap():
            pltpu.sync_copy(pos_hbm.at[pl.ds(pos_base, n_per_w)], rows_v)

        for first in range(pos_gran, n_per_w, pos_gran):
            @pl.when(pos_base == seq - first)
            def _pos_wrap(first=first):
                pltpu.sync_copy(
                    pos_hbm.at[pl.ds(seq - first, first)],
                    rows_v.at[pl.ds(0, first)],
                )
                pltpu.sync_copy(
                    pos_hbm.at[pl.ds(0, n_per_w - first)],
                    rows_v.at[pl.ds(first, n_per_w - first)],
                )

        sems = [sa, sb]
        dvecs = [lax.iota(jnp.int32, 16) + 16 * c for c in range(dim // 16)]
        nq = n_per_w // 4
        vs = [idx_v[pl.ds(g * 16, 16)] for g in range(n_per_w // 16)]

        def fire(q):
            slab = q % 2
            for j in range(4):
                i = 4 * q + j
                v = vs[i // 16][i % 16]
                vb = pl.multiple_of((v >> 7) * 128, 128)
                pltpu.async_copy(
                    tbl_hbm.at[:, pl.ds(vb, 128)],
                    blk_v.at[4 * slab + j],
                    sems[slab],
                )

        def wait_quad(q):
            for _ in range(4):
                pltpu.make_async_copy(
                    tbl_hbm.at[:, pl.ds(0, 128)], blk_v.at[0], sems[q % 2]
                ).wait()

        def extract(q):
            slab = q % 2
            for j in range(4):
                i = 4 * q + j
                v = vs[i // 16][i % 16]
                p = jnp.full((16,), v & 127, jnp.int32)
                for c in range(dim // 16):
                    tok = plsc.load_gather(blk_v.at[4 * slab + j], [dvecs[c], p])
                    sl = (i, pl.ds(16 * c, 16))
                    rows_v[sl] = rows_v[sl] + tok

        fire(0)
        fire(1)
        for q in range(nq):
            wait_quad(q)
            extract(q)
            if q + 2 < nq:
                fire(q + 2)

        pltpu.sync_copy(rows_v, out_hbm.at[pl.ds(base, n_per_w)])

    return _emb(xf, token_table.T, pos_table)


def _tc_half(idx_tc, tbl_t, pos_table, n_sc):
    dim, vocab = tbl_t.shape
    seq, _ = pos_table.shape
    n_tc = idx_tc.shape[0]
    K = 16
    steps = n_tc // K
    assert n_tc % K == 0 and seq % K == 0 and n_sc % K == 0

    def tbl_map(j, i, idx_ref):
        return (0, idx_ref[i * K + j] >> 7)

    def body(idx_sref, *refs):
        blks = refs[:K]
        pos_ref, out_ref = refs[K], refs[K + 1]
        i = pl.program_id(0)
        cols = []
        for j in range(K):
            p = idx_sref[i * K + j] & 127
            rolled = pltpu.roll(blks[j][...], -p, 1)
            cols.append(rolled[:, 0:1])
        ext = jnp.concatenate(cols, axis=1)
        out_ref[...] = ext.T + pos_ref[...]

    grid_spec = pltpu.PrefetchScalarGridSpec(
        num_scalar_prefetch=1,
        grid=(steps,),
        in_specs=[
            *[
                pl.BlockSpec((dim, 128), functools.partial(tbl_map, j))
                for j in range(K)
            ],
            pl.BlockSpec(
                (K, dim),
                lambda i, idx_ref: ((n_sc // K + i) % (seq // K), 0),
            ),
        ],
        out_specs=pl.BlockSpec((K, dim), lambda i, idx_ref: (i, 0)),
    )
    return pl.pallas_call(
        body,
        grid_spec=grid_spec,
        out_shape=jax.ShapeDtypeStruct((n_tc, dim), jnp.float32),
    )(idx_tc, *([tbl_t] * K), pos_table)


def kernel(x, token_table, pos_table):
    batch, seq = x.shape
    vocab, dim = token_table.shape
    n = batch * seq
    n_sc = 13 * n // 16

    xf = x.reshape(n)
    out_sc = _sc_half(xf, token_table, pos_table, n_sc)
    out_tc = _tc_half(xf[n_sc:], token_table.T, pos_table, n_sc)
    out = jnp.concatenate([out_sc, out_tc], axis=0)
    return out.reshape(batch, seq, dim)
